# baseline (speedup 1.0000x reference)
import functools

import jax
import jax.numpy as jnp
from jax import lax
from jax.experimental import pallas as pl
from jax.experimental.pallas import tpu as pltpu
from jax.experimental.pallas import tpu_sc as plsc

N = 10000
E = 320000
D = 128
GAMMA = 0.1
EPS = 0.1
HID = 64
OUT = 128

NC = 2
NS = 16
NW = NC * NS
EW = E // NW
B = 80
CH = EW // B
RPS = 624
TAIL = N - NS * RPS
ZR = 48
NBUF = 3

BM = 1000


def _sc_agg_body(x_hbm, cmb_hbm, out_hbm,
                 pk0_v, pk1_v, pk2_v, dst0_v, dst1_v, dst2_v,
                 msg0_v, msg1_v, msg2_v, agg_sh,
                 is0, is1, is2, gs0, gs1, gs2, ssem):
    c = lax.axis_index("c")
    s = lax.axis_index("s")
    wid = s * NC + c
    bufs = (msg0_v, msg1_v, msg2_v)
    pkb = (pk0_v, pk1_v, pk2_v)
    dstb = (dst0_v, dst1_v, dst2_v)
    isems = (is0, is1, is2)
    gsems = (gs0, gs1, gs2)

    def start_idx(j, b):
        return pltpu.async_copy(
            cmb_hbm.at[pl.ds(wid * EW + j * B, B)], pkb[b], isems[b])

    def wait_idx(j, b):
        pltpu.make_async_copy(
            cmb_hbm.at[pl.ds(wid * EW + j * B, B)], pkb[b], isems[b]).wait()

    def unpack(b):
        for q in range(B // 16):
            v = pkb[b][pl.ds(q * 16, 16)]
            pkb[b][pl.ds(q * 16, 16)] = jax.lax.shift_right_logical(v, 14)
            dstb[b][pl.ds(q * 16, 16)] = jax.lax.bitwise_and(v, 16383)

    def start_gather(b):
        return pltpu.async_copy(x_hbm.at[pkb[b]], bufs[b], gsems[b])

    def wait_gather(b):
        pltpu.make_async_copy(x_hbm.at[pkb[b]], bufs[b], gsems[b]).wait()

    def start_scatter(b):
        return pltpu.async_copy(bufs[b], agg_sh.at[dstb[b]], ssem,
                                add=True)

    for b in range(NBUF):
        start_idx(b, b)
    for b in range(1, NBUF):
        wait_idx(b, b)
        unpack(b)
        start_gather(b)

    z16 = jnp.zeros((16,), jnp.float32)
    for i in range(B):
        for q in range(D // 16):
            msg0_v[i, pl.ds(q * 16, 16)] = z16
    for k in range(RPS // B):
        pltpu.sync_copy(msg0_v, agg_sh.at[pl.ds(s * RPS + k * B, B)])
    pltpu.sync_copy(msg0_v.at[pl.ds(0, RPS - (RPS // B) * B)],
                    agg_sh.at[pl.ds(s * RPS + (RPS // B) * B,
                                    RPS - (RPS // B) * B)])

    @pl.when(s == 0)
    def _zero_tail():
        pltpu.sync_copy(msg0_v.at[pl.ds(0, TAIL)],
                        agg_sh.at[pl.ds(NS * RPS, TAIL)])

    wait_idx(0, 0)
    unpack(0)
    start_gather(0)
    plsc.subcore_barrier()

    def ring(jj, carry):
        j0 = jj * NBUF
        for b in range(NBUF):
            j = j0 + b
            wait_gather(b)

            @pl.when(j + NBUF < CH)
            def _refill():
                start_idx(j + NBUF, b)

            start_scatter(b).wait()

            @pl.when(j + NBUF < CH)
            def _next():
                wait_idx(j + NBUF, b)
                unpack(b)
                start_gather(b)
        return carry

    lax.fori_loop(0, CH // NBUF, ring, 0)
    for j in range(CH - CH % NBUF, CH):
        b = j % NBUF
        wait_gather(b)
        start_scatter(b).wait()

    plsc.subcore_barrier()
    pltpu.sync_copy(agg_sh.at[pl.ds(s * RPS, RPS)],
                    out_hbm.at[c, pl.ds(s * RPS, RPS)])

    @pl.when(s == 0)
    def _write_tail():
        pltpu.sync_copy(agg_sh.at[pl.ds(NS * RPS, TAIL)],
                        out_hbm.at[c, pl.ds(NS * RPS, TAIL)])


_sc_agg = pl.kernel(
    _sc_agg_body,
    out_type=jax.ShapeDtypeStruct((NC, N, D), jnp.float32),
    mesh=plsc.VectorSubcoreMesh(core_axis_name="c", subcore_axis_name="s"),
    scratch_types=[
        pltpu.VMEM((B,), jnp.int32),
        pltpu.VMEM((B,), jnp.int32),
        pltpu.VMEM((B,), jnp.int32),
        pltpu.VMEM((B,), jnp.int32),
        pltpu.VMEM((B,), jnp.int32),
        pltpu.VMEM((B,), jnp.int32),
        pltpu.VMEM((B, D), jnp.float32),
        pltpu.VMEM((B, D), jnp.float32),
        pltpu.VMEM((B, D), jnp.float32),
        pltpu.VMEM_SHARED((N, D), jnp.float32),
        pltpu.SemaphoreType.DMA,
        pltpu.SemaphoreType.DMA,
        pltpu.SemaphoreType.DMA,
        pltpu.SemaphoreType.DMA,
        pltpu.SemaphoreType.DMA,
        pltpu.SemaphoreType.DMA,
        pltpu.SemaphoreType.DMA,
    ],
)


def _dot_nt(a, w):
    return lax.dot_general(a, w, (((1,), (1,)), ((), ())),
                           preferred_element_type=jnp.float32)


def _dot_nn(a, w):
    return lax.dot_general(a, w, (((1,), (0,)), ((), ())),
                           preferred_element_type=jnp.float32)


def _tc_body(x_ref, agg_ref, w_ref, lw_ref, b_ref, r1w_ref, r1b_ref,
             r2w_ref, r2b_ref, o_ref):
    xb = x_ref[...]
    w = w_ref[...]
    agg = agg_ref[0] + agg_ref[1]
    conv = _dot_nt(xb, w) - _dot_nn(xb, w) - GAMMA * xb
    conv = conv + _dot_nt(agg, lw_ref[...])
    h = xb + EPS * jnp.tanh(conv + b_ref[...])
    r = _dot_nt(h, r1w_ref[...]) + r1b_ref[...]
    r = jnp.where(r > 0, r, 0.01 * r)
    r = _dot_nt(r, r2w_ref[...]) + r2b_ref[...]
    o_ref[...] = jnp.where(r > 0, r, 0.01 * r)


def _row_spec(d):
    return pl.BlockSpec((BM, d), lambda i: (i, 0))


def _full_spec(*shape):
    return pl.BlockSpec(shape, lambda i: (0,) * len(shape))


_tc_fused = pl.pallas_call(
    _tc_body,
    grid=(N // BM,),
    in_specs=[_row_spec(D),
              pl.BlockSpec((NC, BM, D), lambda i: (0, i, 0)),
              _full_spec(D, D), _full_spec(D, D), _full_spec(1, D),
              _full_spec(HID, D), _full_spec(1, HID),
              _full_spec(OUT, HID), _full_spec(1, OUT)],
    out_specs=_row_spec(OUT),
    out_shape=jax.ShapeDtypeStruct((N, OUT), jnp.float32),
)


def kernel(x, edge_index, batch, W, b, lin_W, r1_W, r1_b, r2_W, r2_b):
    del batch
    src = edge_index[0].astype(jnp.int32)
    dst = edge_index[1].astype(jnp.int32)
    cmb = (src << 14) | dst

    agg2 = _sc_agg(x, cmb)
    return _tc_fused(x, agg2, W, lin_W, b.reshape(1, D),
                     r1_W, r1_b.reshape(1, HID),
                     r2_W, r2_b.reshape(1, OUT))


# speedup vs baseline: 1.0280x; 1.0280x over previous
import functools

import jax
import jax.numpy as jnp
from jax import lax
from jax.experimental import pallas as pl
from jax.experimental.pallas import tpu as pltpu
from jax.experimental.pallas import tpu_sc as plsc

N = 10000
E = 320000
D = 128
GAMMA = 0.1
EPS = 0.1
HID = 64
OUT = 128

NC = 2
NS = 16
NW = NC * NS
EW = E // NW
B = 80
CH = EW // B
RPS = 624
TAIL = N - NS * RPS
ZR = 48
NBUF = 3

BM = 2000


def _sc_agg_body(x_hbm, cmb_hbm, out_hbm,
                 pk0_v, pk1_v, pk2_v, dst0_v, dst1_v, dst2_v,
                 msg0_v, msg1_v, msg2_v, agg_sh,
                 is0, is1, is2, gs0, gs1, gs2, ssem):
    c = lax.axis_index("c")
    s = lax.axis_index("s")
    wid = s * NC + c
    bufs = (msg0_v, msg1_v, msg2_v)
    pkb = (pk0_v, pk1_v, pk2_v)
    dstb = (dst0_v, dst1_v, dst2_v)
    isems = (is0, is1, is2)
    gsems = (gs0, gs1, gs2)

    def start_idx(j, b):
        return pltpu.async_copy(
            cmb_hbm.at[pl.ds(wid * EW + j * B, B)], pkb[b], isems[b])

    def wait_idx(j, b):
        pltpu.make_async_copy(
            cmb_hbm.at[pl.ds(wid * EW + j * B, B)], pkb[b], isems[b]).wait()

    def unpack(b):
        for q in range(B // 16):
            v = pkb[b][pl.ds(q * 16, 16)]
            pkb[b][pl.ds(q * 16, 16)] = jax.lax.shift_right_logical(v, 14)
            dstb[b][pl.ds(q * 16, 16)] = jax.lax.bitwise_and(v, 16383)

    def start_gather(b):
        return pltpu.async_copy(x_hbm.at[pkb[b]], bufs[b], gsems[b])

    def wait_gather(b):
        pltpu.make_async_copy(x_hbm.at[pkb[b]], bufs[b], gsems[b]).wait()

    def start_scatter(b):
        return pltpu.async_copy(bufs[b], agg_sh.at[dstb[b]], ssem,
                                add=True)

    for b in range(NBUF):
        start_idx(b, b)
    for b in range(1, NBUF):
        wait_idx(b, b)
        unpack(b)
        start_gather(b)

    z16 = jnp.zeros((16,), jnp.float32)
    for i in range(B):
        for q in range(D // 16):
            msg0_v[i, pl.ds(q * 16, 16)] = z16
    for k in range(RPS // B):
        pltpu.sync_copy(msg0_v, agg_sh.at[pl.ds(s * RPS + k * B, B)])
    pltpu.sync_copy(msg0_v.at[pl.ds(0, RPS - (RPS // B) * B)],
                    agg_sh.at[pl.ds(s * RPS + (RPS // B) * B,
                                    RPS - (RPS // B) * B)])

    @pl.when(s == 0)
    def _zero_tail():
        pltpu.sync_copy(msg0_v.at[pl.ds(0, TAIL)],
                        agg_sh.at[pl.ds(NS * RPS, TAIL)])

    wait_idx(0, 0)
    unpack(0)
    start_gather(0)
    plsc.subcore_barrier()

    def ring(jj, carry):
        j0 = jj * NBUF
        for b in range(NBUF):
            j = j0 + b
            wait_gather(b)

            @pl.when(j + NBUF < CH)
            def _refill():
                start_idx(j + NBUF, b)

            start_scatter(b).wait()

            @pl.when(j + NBUF < CH)
            def _next():
                wait_idx(j + NBUF, b)
                unpack(b)
                start_gather(b)
        return carry

    lax.fori_loop(0, CH // NBUF, ring, 0)
    for j in range(CH - CH % NBUF, CH):
        b = j % NBUF
        wait_gather(b)
        start_scatter(b).wait()

    plsc.subcore_barrier()
    pltpu.sync_copy(agg_sh.at[pl.ds(s * RPS, RPS)],
                    out_hbm.at[c, pl.ds(s * RPS, RPS)])

    @pl.when(s == 0)
    def _write_tail():
        pltpu.sync_copy(agg_sh.at[pl.ds(NS * RPS, TAIL)],
                        out_hbm.at[c, pl.ds(NS * RPS, TAIL)])


_sc_agg = pl.kernel(
    _sc_agg_body,
    out_type=jax.ShapeDtypeStruct((NC, N, D), jnp.float32),
    mesh=plsc.VectorSubcoreMesh(core_axis_name="c", subcore_axis_name="s"),
    scratch_types=[
        pltpu.VMEM((B,), jnp.int32),
        pltpu.VMEM((B,), jnp.int32),
        pltpu.VMEM((B,), jnp.int32),
        pltpu.VMEM((B,), jnp.int32),
        pltpu.VMEM((B,), jnp.int32),
        pltpu.VMEM((B,), jnp.int32),
        pltpu.VMEM((B, D), jnp.float32),
        pltpu.VMEM((B, D), jnp.float32),
        pltpu.VMEM((B, D), jnp.float32),
        pltpu.VMEM_SHARED((N, D), jnp.float32),
        pltpu.SemaphoreType.DMA,
        pltpu.SemaphoreType.DMA,
        pltpu.SemaphoreType.DMA,
        pltpu.SemaphoreType.DMA,
        pltpu.SemaphoreType.DMA,
        pltpu.SemaphoreType.DMA,
        pltpu.SemaphoreType.DMA,
    ],
)


def _dot_nt(a, w):
    return lax.dot_general(a, w, (((1,), (1,)), ((), ())),
                           preferred_element_type=jnp.float32)


def _dot_nn(a, w):
    return lax.dot_general(a, w, (((1,), (0,)), ((), ())),
                           preferred_element_type=jnp.float32)


def _tc_body(x_ref, agg_ref, w_ref, lw_ref, b_ref, r1w_ref, r1b_ref,
             r2w_ref, r2b_ref, o_ref):
    xb = x_ref[...]
    w = w_ref[...]
    agg = agg_ref[0] + agg_ref[1]
    conv = _dot_nt(xb, w) - _dot_nn(xb, w) - GAMMA * xb
    conv = conv + _dot_nt(agg, lw_ref[...])
    h = xb + EPS * jnp.tanh(conv + b_ref[...])
    r = _dot_nt(h, r1w_ref[...]) + r1b_ref[...]
    r = jnp.where(r > 0, r, 0.01 * r)
    r = _dot_nt(r, r2w_ref[...]) + r2b_ref[...]
    o_ref[...] = jnp.where(r > 0, r, 0.01 * r)


def _row_spec(d):
    return pl.BlockSpec((BM, d), lambda i: (i, 0))


def _full_spec(*shape):
    return pl.BlockSpec(shape, lambda i: (0,) * len(shape))


_tc_fused = pl.pallas_call(
    _tc_body,
    grid=(N // BM,),
    in_specs=[_row_spec(D),
              pl.BlockSpec((NC, BM, D), lambda i: (0, i, 0)),
              _full_spec(D, D), _full_spec(D, D), _full_spec(1, D),
              _full_spec(HID, D), _full_spec(1, HID),
              _full_spec(OUT, HID), _full_spec(1, OUT)],
    out_specs=_row_spec(OUT),
    out_shape=jax.ShapeDtypeStruct((N, OUT), jnp.float32),
)


def kernel(x, edge_index, batch, W, b, lin_W, r1_W, r1_b, r2_W, r2_b):
    del batch
    src = edge_index[0].astype(jnp.int32)
    dst = edge_index[1].astype(jnp.int32)
    cmb = (src << 14) | dst

    agg2 = _sc_agg(x, cmb)
    return _tc_fused(x, agg2, W, lin_W, b.reshape(1, D),
                     r1_W, r1_b.reshape(1, HID),
                     r2_W, r2_b.reshape(1, OUT))


# speedup vs baseline: 1.0305x; 1.0024x over previous
import functools

import jax
import jax.numpy as jnp
from jax import lax
from jax.experimental import pallas as pl
from jax.experimental.pallas import tpu as pltpu
from jax.experimental.pallas import tpu_sc as plsc

N = 10000
E = 320000
D = 128
GAMMA = 0.1
EPS = 0.1
HID = 64
OUT = 128

NC = 2
NS = 16
NW = NC * NS
EW = E // NW
B = 80
CH = EW // B
RPS = 624
TAIL = N - NS * RPS
ZR = 48
NBUF = 3

BM = 2000


def _sc_agg_body(x_hbm, cmb_hbm, out_hbm,
                 pk0_v, pk1_v, pk2_v, dst0_v, dst1_v, dst2_v,
                 msg0_v, msg1_v, msg2_v, agg_sh,
                 is0, is1, is2, gs0, gs1, gs2, ssem):
    c = lax.axis_index("c")
    s = lax.axis_index("s")
    wid = s * NC + c
    bufs = (msg0_v, msg1_v, msg2_v)
    pkb = (pk0_v, pk1_v, pk2_v)
    dstb = (dst0_v, dst1_v, dst2_v)
    isems = (is0, is1, is2)
    gsems = (gs0, gs1, gs2)

    def start_idx(j, b):
        return pltpu.async_copy(
            cmb_hbm.at[pl.ds(wid * EW + j * B, B)], pkb[b], isems[b])

    def wait_idx(j, b):
        pltpu.make_async_copy(
            cmb_hbm.at[pl.ds(wid * EW + j * B, B)], pkb[b], isems[b]).wait()

    def unpack(b):
        for q in range(B // 16):
            v = pkb[b][pl.ds(q * 16, 16)]
            pkb[b][pl.ds(q * 16, 16)] = jax.lax.shift_right_logical(v, 14)
            dstb[b][pl.ds(q * 16, 16)] = jax.lax.bitwise_and(v, 16383)

    def start_gather(b):
        return pltpu.async_copy(x_hbm.at[pkb[b]], bufs[b], gsems[b])

    def wait_gather(b):
        pltpu.make_async_copy(x_hbm.at[pkb[b]], bufs[b], gsems[b]).wait()

    def start_scatter(b):
        return pltpu.async_copy(bufs[b], agg_sh.at[dstb[b]], ssem,
                                add=True)

    for b in range(NBUF):
        start_idx(b, b)
    for b in range(1, NBUF):
        wait_idx(b, b)
        unpack(b)
        start_gather(b)

    z16 = jnp.zeros((16,), jnp.float32)
    for i in range(B):
        for q in range(D // 16):
            msg0_v[i, pl.ds(q * 16, 16)] = z16
    zcps = [pltpu.async_copy(msg0_v, agg_sh.at[pl.ds(s * RPS + k * B, B)],
                             ssem)
            for k in range(RPS // B)]
    zcps.append(pltpu.async_copy(
        msg0_v.at[pl.ds(0, RPS - (RPS // B) * B)],
        agg_sh.at[pl.ds(s * RPS + (RPS // B) * B, RPS - (RPS // B) * B)],
        ssem))

    @pl.when(s == 0)
    def _zero_tail():
        pltpu.sync_copy(msg0_v.at[pl.ds(0, TAIL)],
                        agg_sh.at[pl.ds(NS * RPS, TAIL)])

    for cp in zcps:
        cp.wait()
    wait_idx(0, 0)
    unpack(0)
    start_gather(0)
    plsc.subcore_barrier()

    def ring(jj, carry):
        j0 = jj * NBUF
        for b in range(NBUF):
            j = j0 + b
            wait_gather(b)

            @pl.when(j + NBUF < CH)
            def _refill():
                start_idx(j + NBUF, b)

            start_scatter(b).wait()

            @pl.when(j + NBUF < CH)
            def _next():
                wait_idx(j + NBUF, b)
                unpack(b)
                start_gather(b)
        return carry

    lax.fori_loop(0, CH // NBUF, ring, 0)
    for j in range(CH - CH % NBUF, CH):
        b = j % NBUF
        wait_gather(b)
        start_scatter(b).wait()

    plsc.subcore_barrier()
    pltpu.sync_copy(agg_sh.at[pl.ds(s * RPS, RPS)],
                    out_hbm.at[c, pl.ds(s * RPS, RPS)])

    @pl.when(s == 0)
    def _write_tail():
        pltpu.sync_copy(agg_sh.at[pl.ds(NS * RPS, TAIL)],
                        out_hbm.at[c, pl.ds(NS * RPS, TAIL)])


_sc_agg = pl.kernel(
    _sc_agg_body,
    out_type=jax.ShapeDtypeStruct((NC, N, D), jnp.float32),
    mesh=plsc.VectorSubcoreMesh(core_axis_name="c", subcore_axis_name="s"),
    scratch_types=[
        pltpu.VMEM((B,), jnp.int32),
        pltpu.VMEM((B,), jnp.int32),
        pltpu.VMEM((B,), jnp.int32),
        pltpu.VMEM((B,), jnp.int32),
        pltpu.VMEM((B,), jnp.int32),
        pltpu.VMEM((B,), jnp.int32),
        pltpu.VMEM((B, D), jnp.float32),
        pltpu.VMEM((B, D), jnp.float32),
        pltpu.VMEM((B, D), jnp.float32),
        pltpu.VMEM_SHARED((N, D), jnp.float32),
        pltpu.SemaphoreType.DMA,
        pltpu.SemaphoreType.DMA,
        pltpu.SemaphoreType.DMA,
        pltpu.SemaphoreType.DMA,
        pltpu.SemaphoreType.DMA,
        pltpu.SemaphoreType.DMA,
        pltpu.SemaphoreType.DMA,
    ],
)


def _dot_nt(a, w):
    return lax.dot_general(a, w, (((1,), (1,)), ((), ())),
                           preferred_element_type=jnp.float32)


def _dot_nn(a, w):
    return lax.dot_general(a, w, (((1,), (0,)), ((), ())),
                           preferred_element_type=jnp.float32)


def _tc_body(x_ref, agg_ref, w_ref, lw_ref, b_ref, r1w_ref, r1b_ref,
             r2w_ref, r2b_ref, o_ref):
    xb = x_ref[...]
    w = w_ref[...]
    agg = agg_ref[0] + agg_ref[1]
    conv = _dot_nt(xb, w) - _dot_nn(xb, w) - GAMMA * xb
    conv = conv + _dot_nt(agg, lw_ref[...])
    h = xb + EPS * jnp.tanh(conv + b_ref[...])
    r = _dot_nt(h, r1w_ref[...]) + r1b_ref[...]
    r = jnp.where(r > 0, r, 0.01 * r)
    r = _dot_nt(r, r2w_ref[...]) + r2b_ref[...]
    o_ref[...] = jnp.where(r > 0, r, 0.01 * r)


def _row_spec(d):
    return pl.BlockSpec((BM, d), lambda i: (i, 0))


def _full_spec(*shape):
    return pl.BlockSpec(shape, lambda i: (0,) * len(shape))


_tc_fused = pl.pallas_call(
    _tc_body,
    grid=(N // BM,),
    in_specs=[_row_spec(D),
              pl.BlockSpec((NC, BM, D), lambda i: (0, i, 0)),
              _full_spec(D, D), _full_spec(D, D), _full_spec(1, D),
              _full_spec(HID, D), _full_spec(1, HID),
              _full_spec(OUT, HID), _full_spec(1, OUT)],
    out_specs=_row_spec(OUT),
    out_shape=jax.ShapeDtypeStruct((N, OUT), jnp.float32),
)


def kernel(x, edge_index, batch, W, b, lin_W, r1_W, r1_b, r2_W, r2_b):
    del batch
    src = edge_index[0].astype(jnp.int32)
    dst = edge_index[1].astype(jnp.int32)
    cmb = (src << 14) | dst

    agg2 = _sc_agg(x, cmb)
    return _tc_fused(x, agg2, W, lin_W, b.reshape(1, D),
                     r1_W, r1_b.reshape(1, HID),
                     r2_W, r2_b.reshape(1, OUT))
